# baseline (speedup 1.0000x reference)
import functools

import jax
import jax.numpy as jnp
from jax.experimental import pallas as pl
from jax.experimental.pallas import tpu as pltpu

_MODEL_DIM = 4096
_NUM_EXPERTS = 64
_HIDDEN = _NUM_EXPERTS * 8
_TM = 1024
_TH = _TM // 2


def _gate_half(x, w1_ref, w2_ref):
    h = jnp.tanh(
        jax.lax.dot_general(
            x, w1_ref[...], (((1,), (1,)), ((), ())),
            preferred_element_type=jnp.float32,
        )
    )
    logits = jax.lax.dot_general(
        h, w2_ref[...], (((1,), (1,)), ((), ())),
        preferred_element_type=jnp.float32,
    )
    mean = jnp.mean(logits, axis=1, keepdims=True)
    var = jnp.sum((logits - mean) ** 2, axis=1, keepdims=True) / (
        _NUM_EXPERTS - 1
    )
    logits = logits / jnp.sqrt(var)

    col = jax.lax.broadcasted_iota(jnp.int32, logits.shape, 1)
    m1 = jnp.max(logits, axis=1, keepdims=True)
    i1 = jnp.argmax(logits, axis=1)[:, None]
    masked = jnp.where(col == i1, -jnp.inf, logits)
    m2 = jnp.max(masked, axis=1, keepdims=True)
    i2 = jnp.argmax(masked, axis=1)[:, None]

    e = jnp.exp(m2 - m1)
    s = 1.0 + e
    g1 = 1.0 / s
    g2 = e / s

    dense = jnp.where(col == i1, g1, 0.0) + jnp.where(col == i2, g2, 0.0)
    return dense, logits


def _router_tile(x_ref, w1_ref, w2_ref, dense_ref, logits_ref):
    dense_a, logits_a = _gate_half(x_ref[: _TH, :], w1_ref, w2_ref)
    dense_b, logits_b = _gate_half(x_ref[_TH:, :], w1_ref, w2_ref)
    dense_ref[:_TH, :] = dense_a
    logits_ref[:_TH, :] = logits_a
    dense_ref[_TH:, :] = dense_b
    logits_ref[_TH:, :] = logits_b


@functools.partial(jax.jit, static_argnames=())
def kernel(input, W1, W2):
    x = input.astype(jnp.float32)
    n_tokens = x.shape[0]
    grid = (n_tokens // _TM,)
    dense, logits = pl.pallas_call(
        _router_tile,
        grid=grid,
        in_specs=[
            pl.BlockSpec((_TM, _MODEL_DIM), lambda i: (i, 0)),
            pl.BlockSpec((_HIDDEN, _MODEL_DIM), lambda i: (0, 0)),
            pl.BlockSpec((_NUM_EXPERTS, _HIDDEN), lambda i: (0, 0)),
        ],
        out_specs=[
            pl.BlockSpec((_TM, _NUM_EXPERTS), lambda i: (i, 0)),
            pl.BlockSpec((_TM, _NUM_EXPERTS), lambda i: (i, 0)),
        ],
        out_shape=[
            jax.ShapeDtypeStruct((n_tokens, _NUM_EXPERTS), jnp.float32),
            jax.ShapeDtypeStruct((n_tokens, _NUM_EXPERTS), jnp.float32),
        ],
        compiler_params=pltpu.CompilerParams(
            vmem_limit_bytes=63 * 1024 * 1024,
        ),
    )(x, W1, W2)
    return (dense, logits)


# speedup vs baseline: 1.0364x; 1.0364x over previous
import functools

import jax
import jax.numpy as jnp
from jax.experimental import pallas as pl
from jax.experimental.pallas import tpu as pltpu

_MODEL_DIM = 4096
_NUM_EXPERTS = 64
_HIDDEN = _NUM_EXPERTS * 8
_TM = 1024
_TH = _TM // 2
_NBUF = 3
_LOOKAHEAD = 2


def _gate_half(x, w1_ref, w2_ref):
    h = jnp.tanh(
        jax.lax.dot_general(
            x, w1_ref[...], (((1,), (1,)), ((), ())),
            preferred_element_type=jnp.float32,
        )
    )
    logits = jax.lax.dot_general(
        h, w2_ref[...], (((1,), (1,)), ((), ())),
        preferred_element_type=jnp.float32,
    )
    mean = jnp.mean(logits, axis=1, keepdims=True)
    var = jnp.sum((logits - mean) ** 2, axis=1, keepdims=True) / (
        _NUM_EXPERTS - 1
    )
    logits = logits / jnp.sqrt(var)

    col = jax.lax.broadcasted_iota(jnp.int32, logits.shape, 1)
    m1 = jnp.max(logits, axis=1, keepdims=True)
    i1 = jnp.argmax(logits, axis=1)[:, None]
    masked = jnp.where(col == i1, -jnp.inf, logits)
    m2 = jnp.max(masked, axis=1, keepdims=True)
    i2 = jnp.argmax(masked, axis=1)[:, None]

    e = jnp.exp(m2 - m1)
    s = 1.0 + e
    g1 = 1.0 / s
    g2 = e / s

    dense = jnp.where(col == i1, g1, 0.0) + jnp.where(col == i2, g2, 0.0)
    return dense, logits


def _router_tile(x_hbm, w1_ref, w2_ref, dense_ref, logits_ref, xbuf, sems):
    i = pl.program_id(0)
    n_tiles = pl.num_programs(0)

    def tile_copy(t):
        slot = jax.lax.rem(t, _NBUF)
        return pltpu.make_async_copy(
            x_hbm.at[pl.ds(t * _TM, _TM), :], xbuf.at[slot], sems.at[slot]
        )

    @pl.when(i == 0)
    def _():
        for b in range(_LOOKAHEAD):
            tile_copy(b).start()

    tile_copy(i).wait()

    @pl.when(i + _LOOKAHEAD < n_tiles)
    def _():
        tile_copy(i + _LOOKAHEAD).start()

    slot = jax.lax.rem(i, _NBUF)
    dense_a, logits_a = _gate_half(xbuf[slot, pl.ds(0, _TH), :], w1_ref, w2_ref)
    dense_b, logits_b = _gate_half(xbuf[slot, pl.ds(_TH, _TH), :], w1_ref, w2_ref)

    dense_ref[:_TH, :] = dense_a
    logits_ref[:_TH, :] = logits_a
    dense_ref[_TH:, :] = dense_b
    logits_ref[_TH:, :] = logits_b


@functools.partial(jax.jit, static_argnames=())
def kernel(input, W1, W2):
    x = input.astype(jnp.float32)
    n_tokens = x.shape[0]
    grid = (n_tokens // _TM,)
    dense, logits = pl.pallas_call(
        _router_tile,
        grid=grid,
        in_specs=[
            pl.BlockSpec(memory_space=pl.ANY),
            pl.BlockSpec((_HIDDEN, _MODEL_DIM), lambda i: (0, 0)),
            pl.BlockSpec((_NUM_EXPERTS, _HIDDEN), lambda i: (0, 0)),
        ],
        out_specs=[
            pl.BlockSpec((_TM, _NUM_EXPERTS), lambda i: (i, 0)),
            pl.BlockSpec((_TM, _NUM_EXPERTS), lambda i: (i, 0)),
        ],
        out_shape=[
            jax.ShapeDtypeStruct((n_tokens, _NUM_EXPERTS), jnp.float32),
            jax.ShapeDtypeStruct((n_tokens, _NUM_EXPERTS), jnp.float32),
        ],
        scratch_shapes=[
            pltpu.VMEM((_NBUF, _TM, _MODEL_DIM), jnp.float32),
            pltpu.SemaphoreType.DMA((_NBUF,)),
        ],
        compiler_params=pltpu.CompilerParams(
            vmem_limit_bytes=63 * 1024 * 1024,
        ),
    )(x, W1, W2)
    return (dense, logits)
